# Initial kernel scaffold; baseline (speedup 1.0000x reference)
import jax
import jax.numpy as jnp
from jax.experimental import pallas as pl


def kernel(rho, c, w, l, roads, lon_idx, lat_idx, W1, b1, W2, b2, Wf1, bf1, W21, b21, W22, b22):
    raise NotImplementedError("write your pallas kernel here")


# speedup vs baseline: 2.8278x; 2.8278x over previous
import functools

import jax
import jax.numpy as jnp
from jax import lax
from jax.experimental import pallas as pl
from jax.experimental.pallas import tpu as pltpu
from jax.experimental.pallas import tpu_sc as plsc

B, S, D_RHO, D_C, HID = 16, 2048, 256, 128, 512
GRID = 17
NBINS = GRID * GRID
NBINS_PAD = 304
HALF = S // 2
CHUNKS = HALF // 16

_SELU_ALPHA = 1.6732632423543772
_SELU_SCALE = 1.0507009873554805


def _sc_histogram(lat_idx, lon_idx):
    mesh = plsc.VectorSubcoreMesh(core_axis_name="c", subcore_axis_name="s")

    @functools.partial(
        pl.kernel,
        mesh=mesh,
        out_type=jax.ShapeDtypeStruct((32, NBINS_PAD), jnp.float32),
        compiler_params=pltpu.CompilerParams(needs_layout_passes=False),
        scratch_types=[
            pltpu.VMEM((HALF,), jnp.int32),
            pltpu.VMEM((HALF,), jnp.int32),
            pltpu.VMEM((16 * NBINS_PAD,), jnp.float32),
            pltpu.VMEM((NBINS_PAD,), jnp.float32),
        ],
    )
    def hist(lat_hbm, lon_hbm, out_hbm, lat_v, lon_v, cnt_v, row_v):
        cid = lax.axis_index("c")
        sid = lax.axis_index("s")
        pltpu.sync_copy(lat_hbm.at[sid, pl.ds(cid * HALF, HALF)], lat_v)
        pltpu.sync_copy(lon_hbm.at[sid, pl.ds(cid * HALF, HALF)], lon_v)

        zeros16 = jnp.zeros((16,), jnp.float32)
        for k in range(16 * NBINS_PAD // 16):
            cnt_v[pl.ds(k * 16, 16)] = zeros16

        lane_base = lax.iota(jnp.int32, 16) * NBINS_PAD
        ones16 = jnp.ones((16,), jnp.float32)

        def body(j, carry):
            off = pl.multiple_of(j * 16, 16)
            la = lat_v[pl.ds(off, 16)]
            lo = lon_v[pl.ds(off, 16)]
            binv = la * GRID + lo
            plsc.addupdate_scatter(cnt_v, [lane_base + binv], ones16)
            return carry

        lax.fori_loop(0, CHUNKS, body, 0)

        for k in range(NBINS_PAD // 16):
            acc = cnt_v[pl.ds(k * 16, 16)]
            for ln in range(1, 16):
                acc = acc + cnt_v[pl.ds(ln * NBINS_PAD + k * 16, 16)]
            row_v[pl.ds(k * 16, 16)] = acc
        pltpu.sync_copy(row_v, out_hbm.at[cid * 16 + sid])

    return hist(lat_idx, lon_idx)


def _prep_body(wc_ref, cfp_ref, W1_ref, b1_ref, W2_ref, b2_ref, Wfc_ref,
               bf1_ref, bias_ref):
    counts = wc_ref[0:16, :] + wc_ref[16:32, :]
    mean_c = jnp.dot(counts, cfp_ref[...],
                     preferred_element_type=jnp.float32) * (1.0 / S)
    h = jnp.dot(mean_c, W1_ref[...],
                preferred_element_type=jnp.float32) + b1_ref[...]
    h = _SELU_SCALE * jnp.where(h > 0.0, h, _SELU_ALPHA * (jnp.exp(h) - 1.0))
    c_tr = jnp.dot(h, W2_ref[...],
                   preferred_element_type=jnp.float32) + b2_ref[...]
    bias_ref[...] = jnp.dot(c_tr, Wfc_ref[...],
                            preferred_element_type=jnp.float32) + bf1_ref[...]


def _main_body(rho_ref, w_ref, Wfr_ref, bias_ref, Wmv_ref, lse1_ref, lse2_ref):
    b = pl.program_id(0)
    x = rho_ref[0]
    bias = bias_ref[pl.ds(b, 1), :]
    hf = jnp.dot(x, Wfr_ref[...], preferred_element_type=jnp.float32) + bias
    hf = jnp.maximum(hf, 0.0)
    a = jnp.dot(hf, Wmv_ref[...], preferred_element_type=jnp.float32)
    logw = jnp.log(w_ref[0])
    a1 = a[:, 0:1] + logw
    a2 = a[:, 1:2] + 2.0 * logw
    m1 = jnp.max(a1, axis=0, keepdims=True)
    l1 = m1 + jnp.log(jnp.sum(jnp.exp(a1 - m1), axis=0, keepdims=True))
    m2 = jnp.max(a2, axis=0, keepdims=True)
    l2 = m2 + jnp.log(jnp.sum(jnp.exp(a2 - m2), axis=0, keepdims=True))
    lse1_ref[...] = jnp.broadcast_to(l1[:, :, None], (1, 8, 128))
    lse2_ref[...] = jnp.broadcast_to(l2[:, :, None], (1, 8, 128))


def kernel(rho, c, w, l, roads, lon_idx, lat_idx, W1, b1, W2, b2, Wf1, bf1,
           W21, b21, W22, b22):
    wc = _sc_histogram(lat_idx.astype(jnp.int32), lon_idx.astype(jnp.int32))

    cc = jnp.transpose(jnp.squeeze(c, 0), (1, 2, 0)).reshape(NBINS, D_C)
    cfp = jnp.pad(cc, ((0, NBINS_PAD - NBINS), (0, 0)))

    bias = pl.pallas_call(
        _prep_body,
        out_shape=jax.ShapeDtypeStruct((B, HID), jnp.float32),
    )(wc, cfp, W1, b1.reshape(1, -1), W2, b2.reshape(1, -1),
      Wf1[D_RHO:], bf1.reshape(1, -1))

    Wmv = jnp.concatenate([W21, W22], axis=1)
    lse1, lse2 = pl.pallas_call(
        _main_body,
        grid=(B,),
        in_specs=[
            pl.BlockSpec((1, S, D_RHO), lambda i: (i, 0, 0)),
            pl.BlockSpec((1, S, 1), lambda i: (i, 0, 0)),
            pl.BlockSpec((D_RHO, HID), lambda i: (0, 0)),
            pl.BlockSpec((B, HID), lambda i: (0, 0)),
            pl.BlockSpec((HID, 2), lambda i: (0, 0)),
        ],
        out_specs=[
            pl.BlockSpec((1, 8, 128), lambda i: (i, 0, 0)),
            pl.BlockSpec((1, 8, 128), lambda i: (i, 0, 0)),
        ],
        out_shape=[
            jax.ShapeDtypeStruct((B, 8, 128), jnp.float32),
            jax.ShapeDtypeStruct((B, 8, 128), jnp.float32),
        ],
    )(rho, w[:, :, None], Wf1[:D_RHO], bias, Wmv)

    logm_agg = lse1[:, 0, 0] + b21[0]
    logv_agg = lse2[:, 0, 0] + b22[0]
    logl = jnp.log(l)
    logmu = logl - logm_agg
    loglam = logl - 3.0 * logm_agg - logv_agg
    return (logmu, loglam)


# speedup vs baseline: 3.7744x; 1.3347x over previous
import functools

import jax
import jax.numpy as jnp
from jax import lax
from jax.experimental import pallas as pl
from jax.experimental.pallas import tpu as pltpu
from jax.experimental.pallas import tpu_sc as plsc

B, S, D_RHO, D_C, HID = 16, 2048, 256, 128, 512
GRID = 17
NBINS = GRID * GRID
NBINS_PAD = 304
HALF = S // 2
CHUNKS = HALF // 16

_SELU_ALPHA = 1.6732632423543772
_SELU_SCALE = 1.0507009873554805


def _sc_histogram(lat_idx, lon_idx):
    mesh = plsc.VectorSubcoreMesh(core_axis_name="c", subcore_axis_name="s")

    @functools.partial(
        pl.kernel,
        mesh=mesh,
        out_type=jax.ShapeDtypeStruct((32, NBINS_PAD), jnp.float32),
        compiler_params=pltpu.CompilerParams(needs_layout_passes=False),
        scratch_types=[
            pltpu.VMEM((HALF,), jnp.int32),
            pltpu.VMEM((HALF,), jnp.int32),
            pltpu.VMEM((16 * NBINS_PAD,), jnp.float32),
            pltpu.VMEM((NBINS_PAD,), jnp.float32),
        ],
    )
    def hist(lat_hbm, lon_hbm, out_hbm, lat_v, lon_v, cnt_v, row_v):
        cid = lax.axis_index("c")
        sid = lax.axis_index("s")
        pltpu.sync_copy(lat_hbm.at[sid, pl.ds(cid * HALF, HALF)], lat_v)
        pltpu.sync_copy(lon_hbm.at[sid, pl.ds(cid * HALF, HALF)], lon_v)

        zeros16 = jnp.zeros((16,), jnp.float32)
        for k in range(16 * NBINS_PAD // 16):
            cnt_v[pl.ds(k * 16, 16)] = zeros16

        lane_base = lax.iota(jnp.int32, 16) * NBINS_PAD
        ones16 = jnp.ones((16,), jnp.float32)

        def body(j, carry):
            off = pl.multiple_of(j * 16, 16)
            la = lat_v[pl.ds(off, 16)]
            lo = lon_v[pl.ds(off, 16)]
            binv = la * GRID + lo
            plsc.addupdate_scatter(cnt_v, [lane_base + binv], ones16)
            return carry

        lax.fori_loop(0, CHUNKS, body, 0)

        for k in range(NBINS_PAD // 16):
            acc = cnt_v[pl.ds(k * 16, 16)]
            for ln in range(1, 16):
                acc = acc + cnt_v[pl.ds(ln * NBINS_PAD + k * 16, 16)]
            row_v[pl.ds(k * 16, 16)] = acc
        pltpu.sync_copy(row_v, out_hbm.at[cid * 16 + sid])

    return hist(lat_idx, lon_idx)


def _main_body(rho_ref, w_ref, wc_ref, c2_ref, W1_ref, b1_ref, W2_ref,
               b2_ref, Wf1_ref, Wf1b_ref, bf1_ref, Wmv_ref,
               lse1_ref, lse2_ref, bias_v):
    b = pl.program_id(0)

    @pl.when(b == 0)
    def _prep():
        counts = wc_ref[0:16, 0:NBINS] + wc_ref[16:32, 0:NBINS]
        mean_c = lax.dot_general(
            counts, c2_ref[...], (((1,), (1,)), ((), ())),
            preferred_element_type=jnp.float32) * (1.0 / S)
        h = jnp.dot(mean_c, W1_ref[...],
                    preferred_element_type=jnp.float32) + b1_ref[...]
        h = _SELU_SCALE * jnp.where(h > 0.0, h,
                                    _SELU_ALPHA * (jnp.exp(h) - 1.0))
        c_tr = jnp.dot(h, W2_ref[...],
                       preferred_element_type=jnp.float32) + b2_ref[...]
        bias_v[...] = jnp.dot(c_tr, Wf1_ref[D_RHO:D_RHO + D_C, :],
                              preferred_element_type=jnp.float32) + bf1_ref[...]

    x = rho_ref[0].astype(jnp.bfloat16)
    hf = jnp.dot(x, Wf1b_ref[...],
                 preferred_element_type=jnp.float32)
    hf = jnp.maximum(hf + bias_v[pl.ds(b, 1), :], 0.0)
    aT = lax.dot_general(
        Wmv_ref[...], hf.astype(jnp.bfloat16), (((1,), (1,)), ((), ())),
        preferred_element_type=jnp.float32)
    logw = jnp.log(w_ref[0])
    a1 = aT[0:1, :] + logw
    a2 = aT[1:2, :] + 2.0 * logw
    m1 = jnp.max(a1, axis=1, keepdims=True)
    l1 = m1 + jnp.log(jnp.sum(jnp.exp(a1 - m1), axis=1, keepdims=True))
    m2 = jnp.max(a2, axis=1, keepdims=True)
    l2 = m2 + jnp.log(jnp.sum(jnp.exp(a2 - m2), axis=1, keepdims=True))
    lse1_ref[...] = jnp.broadcast_to(l1[:, :, None], (1, 8, 128))
    lse2_ref[...] = jnp.broadcast_to(l2[:, :, None], (1, 8, 128))


def _run_main(rho, w3, wc, c2, W1, b1, W2, b2, Wf1, Wf1b, bf1, Wmv):
    return pl.pallas_call(
        _main_body,
        grid=(B,),
        in_specs=[
            pl.BlockSpec((1, S, D_RHO), lambda i: (i, 0, 0)),
            pl.BlockSpec((1, 1, S), lambda i: (i, 0, 0)),
            pl.BlockSpec((32, NBINS_PAD), lambda i: (0, 0)),
            pl.BlockSpec((D_C, NBINS), lambda i: (0, 0)),
            pl.BlockSpec((D_C, 2 * D_C), lambda i: (0, 0)),
            pl.BlockSpec((1, 2 * D_C), lambda i: (0, 0)),
            pl.BlockSpec((2 * D_C, D_C), lambda i: (0, 0)),
            pl.BlockSpec((1, D_C), lambda i: (0, 0)),
            pl.BlockSpec((D_RHO + D_C, HID), lambda i: (0, 0)),
            pl.BlockSpec((D_RHO, HID), lambda i: (0, 0)),
            pl.BlockSpec((1, HID), lambda i: (0, 0)),
            pl.BlockSpec((2, HID), lambda i: (0, 0)),
        ],
        out_specs=[
            pl.BlockSpec((1, 8, 128), lambda i: (i, 0, 0)),
            pl.BlockSpec((1, 8, 128), lambda i: (i, 0, 0)),
        ],
        out_shape=[
            jax.ShapeDtypeStruct((B, 8, 128), jnp.float32),
            jax.ShapeDtypeStruct((B, 8, 128), jnp.float32),
        ],
        scratch_shapes=[pltpu.VMEM((B, HID), jnp.float32)],
    )(rho, w3, wc, c2, W1, b1, W2, b2, Wf1, Wf1b, bf1, Wmv)


def kernel(rho, c, w, l, roads, lon_idx, lat_idx, W1, b1, W2, b2, Wf1, bf1,
           W21, b21, W22, b22):
    wc = _sc_histogram(lat_idx.astype(jnp.int32), lon_idx.astype(jnp.int32))

    c2 = c.reshape(D_C, NBINS)
    w3 = w.reshape(B, 1, S)
    Wf1b = Wf1[:D_RHO].astype(jnp.bfloat16)
    Wmv = jnp.concatenate(
        [W21.reshape(1, HID), W22.reshape(1, HID)], axis=0
    ).astype(jnp.bfloat16)

    lse1, lse2 = _run_main(rho, w3, wc, c2, W1, b1.reshape(1, -1), W2,
                           b2.reshape(1, -1), Wf1, Wf1b,
                           bf1.reshape(1, -1), Wmv)

    logm_agg = lse1[:, 0, 0] + b21[0]
    logv_agg = lse2[:, 0, 0] + b22[0]
    logl = jnp.log(l)
    logmu = logl - logm_agg
    loglam = logl - 3.0 * logm_agg - logv_agg
    return (logmu, loglam)
